# baseline (speedup 1.0000x reference)
import functools
import jax
import jax.numpy as jnp
import numpy as np
from jax import lax
from jax.experimental import pallas as pl
from jax.experimental.pallas import tpu as pltpu
from jax.experimental.pallas import tpu_sc as plsc

N = 4096
H = 256
W = 256
C = 3
G = 64
ROWS = 8
NB = H // ROWS
PX = ROWS * W
RCUT = 8.5
IDXC = 128
NPAD = N + IDXC
NCHP = NPAD // G
NV = N // 16
NPAR = 8


def _project_kernel(p_ref, cy_ref, cx_ref, a_ref, b_ref, c_ref,
                    v0_ref, v1_ref, v2_ref, bounds_ref):
    xt = jnp.tanh(p_ref[0:1, :])
    yt = jnp.tanh(p_ref[1:2, :])
    s0 = jnp.abs(p_ref[2:3, :]) + 0.3
    s1 = jnp.abs(p_ref[3:4, :]) + 0.3
    theta = jax.nn.sigmoid(p_ref[4:5, :]) * (2.0 * np.pi)
    cos_t = jnp.cos(theta)
    sin_t = jnp.sin(theta)
    a = cos_t * cos_t * s0 * s0 + sin_t * sin_t * s1 * s1
    b = cos_t * sin_t * (s0 * s0 - s1 * s1)
    c = sin_t * sin_t * s0 * s0 + cos_t * cos_t * s1 * s1
    inv_det = 1.0 / (a * c - b * b)
    cx = 0.5 * W * (xt + 1.0) - 0.5
    cy = 0.5 * H * (yt + 1.0) - 0.5
    op = p_ref[8:9, :]
    cy_ref[...] = cy.reshape(N)
    cx_ref[...] = cx.reshape(N)
    a_ref[...] = (-0.5 * c * inv_det).reshape(N)
    b_ref[...] = (b * inv_det).reshape(N)
    c_ref[...] = (-0.5 * a * inv_det).reshape(N)
    v0_ref[...] = (p_ref[5:6, :] * op).reshape(N)
    v1_ref[...] = (p_ref[6:7, :] * op).reshape(N)
    v2_ref[...] = (p_ref[7:8, :] * op).reshape(N)
    ri = lax.broadcasted_iota(jnp.int32, (NB, 1), 0)
    lo_thr = ((ri - 2) * ROWS).astype(jnp.float32)
    hi_thr = ((ri + 2) * ROWS).astype(jnp.float32)
    lo_cnt = jnp.sum((cy < lo_thr).astype(jnp.int32), axis=1)
    hi_cnt = jnp.sum((cy < hi_thr).astype(jnp.int32), axis=1)
    ri1 = ri.reshape(NB)
    lo_cnt = jnp.where(ri1 >= 3, lo_cnt, 0)
    hi_cnt = jnp.where(ri1 >= NB - 2, N, hi_cnt)
    bounds_ref[0:1, :] = lo_cnt.reshape(1, NB)
    bounds_ref[1:2, :] = hi_cnt.reshape(1, NB)


def _bin_kernel(cy_h, cx_h, a_h, b_h, c_h, v0_h, v1_h, v2_h,
                ocy_h, ocx_h, oa_h, ob_h, oc_h, ov0_h, ov1_h, ov2_h,
                cyv, gid, posb, stage, sem1, sem2):
    wid = lax.axis_index("s") * 2 + lax.axis_index("c")
    ins = (cy_h, cx_h, a_h, b_h, c_h, v0_h, v1_h, v2_h)
    outs = (ocy_h, ocx_h, oa_h, ob_h, oc_h, ov0_h, ov1_h, ov2_h)

    pltpu.sync_copy(cy_h, cyv)

    @pl.when(wid == 0)
    def _():
        zv = jnp.zeros((16,), jnp.float32)
        for q in range(IDXC // 16):
            stage[0, pl.ds(q * 16, 16)] = zv
        for p in range(NPAR):
            pltpu.async_copy(stage.at[0], outs[p].at[pl.ds(N, IDXC)],
                             sem2).wait()

    lane = lax.iota(jnp.int32, 16)

    def bucket_of(v):
        return jnp.clip((v * (1.0 / ROWS)).astype(jnp.int32), 0, NB - 1)

    def count_body(k, carry):
        acc_lt, acc_eq = carry
        bkt = bucket_of(cyv[pl.ds(k * 16, 16)])
        acc_lt = acc_lt + jnp.where(bkt < wid, 1, 0)
        acc_eq = acc_eq + jnp.where(bkt == wid, 1, 0)
        return acc_lt, acc_eq

    zero16 = jnp.zeros((16,), jnp.int32)
    acc_lt, acc_eq = lax.fori_loop(0, NV, count_body, (zero16, zero16))

    def prefix(x):
        for s in (1, 2, 4, 8):
            shifted = x.at[jnp.maximum(lane - s, 0)].get(
                mode="promise_in_bounds")
            x = x + jnp.where(lane >= s, shifted, 0)
        return x

    peq = prefix(acc_eq)
    excl = peq - acc_eq
    plt = prefix(acc_lt)
    base = plt[15]
    n = peq[15]

    def scan_body(k, dest_vec):
        bkt = bucket_of(cyv[pl.ds(k * 16, 16)])
        eq = bkt == wid
        plsc.store_scatter(gid, [dest_vec], k * 16 + lane, mask=eq)
        return dest_vec + jnp.where(eq, 1, 0)

    lax.fori_loop(0, NV, scan_body, excl)

    nch = (n + IDXC - 1) // IDXC

    def dma_body(j, carry):
        for q in range(IDXC // 16):
            g = j * IDXC + q * 16 + lane
            pv = jnp.where(g < n, base + g, N + q * 16 + lane)
            posb[pl.ds(q * 16, 16)] = pv
        gets = [pltpu.async_copy(ins[p].at[gid.at[pl.ds(j * IDXC, IDXC)]],
                                 stage.at[p], sem1)
                for p in range(NPAR)]
        for d in gets:
            d.wait()
        puts = [pltpu.async_copy(stage.at[p], outs[p].at[posb], sem2)
                for p in range(NPAR)]
        for d in puts:
            d.wait()
        return carry

    lax.fori_loop(0, nch, dma_body, jnp.int32(0))


def _bin_call(cy, cx, aq, bq, cq, v0, v1, v2):
    o1 = jax.ShapeDtypeStruct((NPAD,), jnp.float32)
    fn = pl.kernel(
        _bin_kernel,
        mesh=plsc.VectorSubcoreMesh(core_axis_name="c", subcore_axis_name="s"),
        compiler_params=pltpu.CompilerParams(needs_layout_passes=False),
        out_type=[o1] * NPAR,
        scratch_types=[
            pltpu.VMEM((N,), jnp.float32),
            pltpu.VMEM((N,), jnp.int32),
            pltpu.VMEM((IDXC,), jnp.int32),
            pltpu.VMEM((NPAR, IDXC), jnp.float32),
            pltpu.SemaphoreType.DMA,
            pltpu.SemaphoreType.DMA,
        ],
    )
    return fn(cy, cx, aq, bq, cq, v0, v1, v2)


def _raster_kernel(bounds_ref, cx_ref, cy_ref, a_ref, b_ref, c_ref,
                   v0_ref, v1_ref, v2_ref, out_ref):
    i = pl.program_id(0)
    elo = bounds_ref[0, i]
    ehi = bounds_ref[1, i]
    lo = elo // G
    hi = (ehi + G - 1) // G
    pix = lax.broadcasted_iota(jnp.int32, (G, PX), 1)
    gx = (pix & (W - 1)).astype(jnp.float32)
    gy = (pix >> 8).astype(jnp.float32) + (i * ROWS).astype(jnp.float32)

    def chunk(j, acc):
        jm = jnp.minimum(j, NCHP - 1)
        cxc = cx_ref[jm].reshape(G, 1)
        cyc = cy_ref[jm].reshape(G, 1)
        ac = a_ref[jm].reshape(G, 1)
        bc = b_ref[jm].reshape(G, 1)
        cc = c_ref[jm].reshape(G, 1)
        dx = gx - cxc
        dy = gy - cyc
        power = (ac * dx) * dx + ((bc * dx) + (cc * dy)) * dy
        ent = lax.broadcasted_iota(jnp.int32, (G, 1), 0) + j * G
        power = jnp.where(ent < ehi, power, -1e30)
        w = jnp.exp(power)
        vt = jnp.concatenate([v0_ref[jm].reshape(1, G),
                              v1_ref[jm].reshape(1, G),
                              v2_ref[jm].reshape(1, G)], axis=0)
        return acc + jnp.dot(vt, w, preferred_element_type=jnp.float32)

    def body(k, acc):
        j = lo + 2 * k
        return chunk(j + 1, chunk(j, acc))

    npair = (hi - lo + 1) // 2
    acc = lax.fori_loop(0, npair, body, jnp.zeros((C, PX), jnp.float32))
    out_ref[...] = acc.reshape(C, ROWS, W)


def kernel(xy, scaling, rotation, values, opacity):
    packed = jnp.concatenate(
        [xy, scaling, rotation, values, opacity], axis=1).T
    o1 = jax.ShapeDtypeStruct((N,), jnp.float32)
    cy, cx, aq, bq, cq, v0, v1, v2, bounds = pl.pallas_call(
        _project_kernel,
        out_shape=[o1] * 8 + [jax.ShapeDtypeStruct((2, NB), jnp.int32)],
    )(packed)

    bcy, bcx, ba, bb, bc, bv0, bv1, bv2 = _bin_call(
        cy, cx, aq, bq, cq, v0, v1, v2)

    q = lambda x: x.reshape(NCHP, G)
    full = lambda shp: pl.BlockSpec(shp, lambda *_: tuple(0 for _ in shp))
    out = pl.pallas_call(
        _raster_kernel,
        grid_spec=pltpu.PrefetchScalarGridSpec(
            num_scalar_prefetch=1,
            grid=(NB,),
            in_specs=[full((NCHP, G))] * 8,
            out_specs=pl.BlockSpec((C, ROWS, W), lambda i, b: (0, i, 0)),
        ),
        out_shape=jax.ShapeDtypeStruct((C, H, W), jnp.float32),
    )(bounds, q(bcx), q(bcy), q(ba), q(bb), q(bc), q(bv0), q(bv1), q(bv2))

    return out.reshape(1, C, H, W)


# speedup vs baseline: 4.2144x; 4.2144x over previous
import functools
import jax
import jax.numpy as jnp
import numpy as np
from jax import lax
from jax.experimental import pallas as pl
from jax.experimental.pallas import tpu as pltpu
from jax.experimental.pallas import tpu_sc as plsc

N = 4096
H = 256
W = 256
C = 3
G = 64
ROWS = 8
NB = H // ROWS
PX = ROWS * W
RCUT = 8.5
IDXC = 128
NPAD = N + IDXC
NCHP = NPAD // G
NV = N // 16
NPAR = 8


def _project_kernel(p_ref, cy_ref, tab_ref, bounds_ref):
    xt = jnp.tanh(p_ref[0:1, :])
    yt = jnp.tanh(p_ref[1:2, :])
    s0 = jnp.abs(p_ref[2:3, :]) + 0.3
    s1 = jnp.abs(p_ref[3:4, :]) + 0.3
    theta = jax.nn.sigmoid(p_ref[4:5, :]) * (2.0 * np.pi)
    cos_t = jnp.cos(theta)
    sin_t = jnp.sin(theta)
    a = cos_t * cos_t * s0 * s0 + sin_t * sin_t * s1 * s1
    b = cos_t * sin_t * (s0 * s0 - s1 * s1)
    c = sin_t * sin_t * s0 * s0 + cos_t * cos_t * s1 * s1
    inv_det = 1.0 / (a * c - b * b)
    cx = 0.5 * W * (xt + 1.0) - 0.5
    cy = 0.5 * H * (yt + 1.0) - 0.5
    op = p_ref[8:9, :]
    cy_ref[...] = cy.reshape(N)
    rows = jnp.concatenate(
        [cx, cy, -0.5 * c * inv_det, b * inv_det, -0.5 * a * inv_det,
         p_ref[5:6, :] * op, p_ref[6:7, :] * op, p_ref[7:8, :] * op], axis=0)
    tab_ref[...] = rows.T
    ri = lax.broadcasted_iota(jnp.int32, (NB, 1), 0)
    lo_thr = ((ri - 2) * ROWS).astype(jnp.float32)
    hi_thr = ((ri + 2) * ROWS).astype(jnp.float32)
    lo_cnt = jnp.sum((cy < lo_thr).astype(jnp.int32), axis=1)
    hi_cnt = jnp.sum((cy < hi_thr).astype(jnp.int32), axis=1)
    ri1 = ri.reshape(NB)
    lo_cnt = jnp.where(ri1 >= 3, lo_cnt, 0)
    hi_cnt = jnp.where(ri1 >= NB - 2, N, hi_cnt)
    bounds_ref[0:1, :] = lo_cnt.reshape(1, NB)
    bounds_ref[1:2, :] = hi_cnt.reshape(1, NB)


def _bin_kernel(cy_h, tab_h, zeros_h, otab_h, cyv, gid, posb, stage,
                sem1, sem2):
    wid = lax.axis_index("s") * 2 + lax.axis_index("c")

    pltpu.sync_copy(cy_h, cyv)

    @pl.when(wid == 0)
    def _():
        pltpu.sync_copy(zeros_h, stage)
        pltpu.async_copy(stage, otab_h.at[pl.ds(N, IDXC)], sem2).wait()

    def zfill_body(k, carry):
        gid[pl.ds(k * 16, 16)] = jnp.zeros((16,), jnp.int32)
        return carry

    lax.fori_loop(0, NV, zfill_body, jnp.int32(0))

    lane = lax.iota(jnp.int32, 16)

    def bucket_of(v):
        return jnp.clip((v * (1.0 / ROWS)).astype(jnp.int32), 0, NB - 1)

    def count_body(k, carry):
        acc_lt, acc_eq = carry
        bkt = bucket_of(cyv[pl.ds(k * 16, 16)])
        acc_lt = acc_lt + jnp.where(bkt < wid, 1, 0)
        acc_eq = acc_eq + jnp.where(bkt == wid, 1, 0)
        return acc_lt, acc_eq

    zero16 = jnp.zeros((16,), jnp.int32)
    acc_lt, acc_eq = lax.fori_loop(0, NV, count_body, (zero16, zero16))

    def prefix(x):
        for s in (1, 2, 4, 8):
            shifted = x.at[jnp.maximum(lane - s, 0)].get(
                mode="promise_in_bounds")
            x = x + jnp.where(lane >= s, shifted, 0)
        return x

    peq = prefix(acc_eq)
    excl = peq - acc_eq
    plt = prefix(acc_lt)
    base = plt[15]
    n = peq[15]

    def scan_body(k, dest_vec):
        bkt = bucket_of(cyv[pl.ds(k * 16, 16)])
        eq = bkt == wid
        plsc.store_scatter(gid, [dest_vec], k * 16 + lane, mask=eq)
        return dest_vec + jnp.where(eq, 1, 0)

    lax.fori_loop(0, NV, scan_body, excl)

    nch = (n + IDXC - 1) // IDXC

    def dma_body(j, carry):
        for q in range(IDXC // 16):
            g = j * IDXC + q * 16 + lane
            pv = jnp.where(g < n, base + g, N + q * 16 + lane)
            posb[pl.ds(q * 16, 16)] = pv
        pltpu.async_copy(tab_h.at[gid.at[pl.ds(j * IDXC, IDXC)]],
                         stage, sem1).wait()
        pltpu.async_copy(stage, otab_h.at[posb], sem2).wait()
        return carry

    lax.fori_loop(0, nch, dma_body, jnp.int32(0))


def _bin_call(cy, tab, zeros):
    fn = pl.kernel(
        _bin_kernel,
        mesh=plsc.VectorSubcoreMesh(core_axis_name="c", subcore_axis_name="s"),
        compiler_params=pltpu.CompilerParams(needs_layout_passes=False,
                                             use_tc_tiling_on_sc=False),
        out_type=jax.ShapeDtypeStruct((NPAD, NPAR), jnp.float32),
        scratch_types=[
            pltpu.VMEM((N,), jnp.float32),
            pltpu.VMEM((N,), jnp.int32),
            pltpu.VMEM((IDXC,), jnp.int32),
            pltpu.VMEM((IDXC, NPAR), jnp.float32),
            pltpu.SemaphoreType.DMA,
            pltpu.SemaphoreType.DMA,
        ],
    )
    return fn(cy, tab, zeros)


def _raster_kernel(bounds_ref, tab_ref, out_ref):
    i = pl.program_id(0)
    elo = bounds_ref[0, i]
    ehi = bounds_ref[1, i]
    lo = elo // G
    hi = (ehi + G - 1) // G
    pix = lax.broadcasted_iota(jnp.int32, (G, PX), 1)
    gx = (pix & (W - 1)).astype(jnp.float32)
    gy = (pix >> 8).astype(jnp.float32) + (i * ROWS).astype(jnp.float32)

    def chunk(j, acc):
        jm = jnp.minimum(j, NCHP - 1)
        slab = tab_ref[jm]
        cxc = slab[:, 0:1]
        cyc = slab[:, 1:2]
        ac = slab[:, 2:3]
        bc = slab[:, 3:4]
        cc = slab[:, 4:5]
        dx = gx - cxc
        dy = gy - cyc
        power = (ac * dx) * dx + ((bc * dx) + (cc * dy)) * dy
        ent = lax.broadcasted_iota(jnp.int32, (G, 1), 0) + j * G
        power = jnp.where(ent < ehi, power, -1e30)
        w = jnp.exp(power)
        vt = slab[:, 5:8]
        return acc + lax.dot_general(
            vt, w, (((0,), (0,)), ((), ())),
            preferred_element_type=jnp.float32)

    def body(k, acc):
        j = lo + 2 * k
        return chunk(j + 1, chunk(j, acc))

    npair = (hi - lo + 1) // 2
    acc = lax.fori_loop(0, npair, body, jnp.zeros((C, PX), jnp.float32))
    out_ref[...] = acc.reshape(C, ROWS, W)


def kernel(xy, scaling, rotation, values, opacity):
    packed = jnp.concatenate(
        [xy, scaling, rotation, values, opacity], axis=1).T
    cy, tab, bounds = pl.pallas_call(
        _project_kernel,
        out_shape=[jax.ShapeDtypeStruct((N,), jnp.float32),
                   jax.ShapeDtypeStruct((N, NPAR), jnp.float32),
                   jax.ShapeDtypeStruct((2, NB), jnp.int32)],
    )(packed)

    btab = _bin_call(cy, tab, jnp.zeros((IDXC, NPAR), jnp.float32))

    full = lambda shp: pl.BlockSpec(shp, lambda *_: tuple(0 for _ in shp))
    out = pl.pallas_call(
        _raster_kernel,
        grid_spec=pltpu.PrefetchScalarGridSpec(
            num_scalar_prefetch=1,
            grid=(NB,),
            in_specs=[full((NCHP, G, NPAR))],
            out_specs=pl.BlockSpec((C, ROWS, W), lambda i, b: (0, i, 0)),
        ),
        out_shape=jax.ShapeDtypeStruct((C, H, W), jnp.float32),
    )(bounds, btab.reshape(NCHP, G, NPAR))

    return out.reshape(1, C, H, W)


# speedup vs baseline: 6.6786x; 1.5847x over previous
import jax
import jax.numpy as jnp
import numpy as np
from jax import lax
from jax.experimental import pallas as pl
from jax.experimental.pallas import tpu as pltpu

N = 4096
H = 256
W = 256
C = 3
G = 64
ROWS = 8
NB = H // ROWS
NCH = N // G
PX = ROWS * W
RCUT = 8.5


def _project_kernel(p_ref, cy_ref, cx_ref, a_ref, b_ref, c_ref,
                    v0_ref, v1_ref, v2_ref, bounds_ref):
    xt = jnp.tanh(p_ref[0:1, :])
    yt = jnp.tanh(p_ref[1:2, :])
    s0 = jnp.abs(p_ref[2:3, :]) + 0.3
    s1 = jnp.abs(p_ref[3:4, :]) + 0.3
    theta = jax.nn.sigmoid(p_ref[4:5, :]) * (2.0 * np.pi)
    cos_t = jnp.cos(theta)
    sin_t = jnp.sin(theta)
    a = cos_t * cos_t * s0 * s0 + sin_t * sin_t * s1 * s1
    b = cos_t * sin_t * (s0 * s0 - s1 * s1)
    c = sin_t * sin_t * s0 * s0 + cos_t * cos_t * s1 * s1
    inv_det = 1.0 / (a * c - b * b)
    cx = 0.5 * W * (xt + 1.0) - 0.5
    cy = 0.5 * H * (yt + 1.0) - 0.5
    op = p_ref[8:9, :]
    cy_ref[...] = cy.reshape(N)
    cx_ref[...] = cx.reshape(N)
    a_ref[...] = (-0.5 * c * inv_det).reshape(N)
    b_ref[...] = (b * inv_det).reshape(N)
    c_ref[...] = (-0.5 * a * inv_det).reshape(N)
    v0_ref[...] = (p_ref[5:6, :] * op).reshape(N)
    v1_ref[...] = (p_ref[6:7, :] * op).reshape(N)
    v2_ref[...] = (p_ref[7:8, :] * op).reshape(N)
    ri = lax.broadcasted_iota(jnp.int32, (NB, 1), 0)
    rowlo = (ri * ROWS).astype(jnp.float32)
    lo_cnt = jnp.sum((cy < rowlo - RCUT).astype(jnp.int32), axis=1)
    hi_cnt = jnp.sum((cy <= rowlo + (ROWS - 1) + RCUT).astype(jnp.int32),
                     axis=1)
    bounds_ref[0:1, :] = lo_cnt.reshape(1, NB)
    bounds_ref[1:2, :] = hi_cnt.reshape(1, NB)


def _raster_kernel(bounds_ref, cx_ref, cy_ref, a_ref, b_ref, c_ref, vt_ref,
                   out_ref):
    i = pl.program_id(0)
    elo = bounds_ref[0, i]
    ehi = bounds_ref[1, i]
    lo = elo // G
    hi = (ehi + G - 1) // G
    pix = lax.broadcasted_iota(jnp.int32, (G, PX), 1)
    gx = (pix & (W - 1)).astype(jnp.float32)
    gy = (pix >> 8).astype(jnp.float32) + (i * ROWS).astype(jnp.float32)

    def chunk(j, acc):
        jm = jnp.minimum(j, NCH - 1)
        cxc = cx_ref[jm].reshape(G, 1)
        cyc = cy_ref[jm].reshape(G, 1)
        ac = a_ref[jm].reshape(G, 1)
        bc = b_ref[jm].reshape(G, 1)
        cc = c_ref[jm].reshape(G, 1)
        dx = gx - cxc
        dy = gy - cyc
        power = (ac * dx) * dx + ((bc * dx) + (cc * dy)) * dy
        ent = lax.broadcasted_iota(jnp.int32, (G, 1), 0) + j * G
        live = (ent < ehi).astype(jnp.float32)
        w = jnp.exp(power) * live
        return acc + jnp.dot(vt_ref[:, jm], w, preferred_element_type=jnp.float32)

    def body(k, acc):
        j = lo + 2 * k
        return chunk(j + 1, chunk(j, acc))

    npair = (hi - lo + 1) // 2
    acc = lax.fori_loop(0, npair, body, jnp.zeros((C, PX), jnp.float32))
    out_ref[...] = acc.reshape(C, ROWS, W)


def kernel(xy, scaling, rotation, values, opacity):
    packed = jnp.concatenate(
        [xy, scaling, rotation, values, opacity], axis=1).T
    o1 = jax.ShapeDtypeStruct((N,), jnp.float32)
    cy, cx, aq, bq, cq, v0, v1, v2, bounds = pl.pallas_call(
        _project_kernel,
        out_shape=[o1] * 8 + [jax.ShapeDtypeStruct((2, NB), jnp.int32)],
    )(packed)

    cy_s, cx_s, a_s, b_s, c_s, v0, v1, v2 = lax.sort(
        (cy, cx, aq, bq, cq, v0, v1, v2), num_keys=1)

    q = lambda x: x.reshape(NCH, G)
    vt = jnp.stack([v0, v1, v2]).reshape(C, NCH, G)

    full = lambda shp: pl.BlockSpec(shp, lambda *_: tuple(0 for _ in shp))
    out = pl.pallas_call(
        _raster_kernel,
        grid_spec=pltpu.PrefetchScalarGridSpec(
            num_scalar_prefetch=1,
            grid=(NB,),
            in_specs=[full((NCH, G))] * 5 + [full((C, NCH, G))],
            out_specs=pl.BlockSpec((C, ROWS, W), lambda i, b: (0, i, 0)),
        ),
        out_shape=jax.ShapeDtypeStruct((C, H, W), jnp.float32),
    )(bounds, q(cx_s), q(cy_s), q(a_s), q(b_s), q(c_s), vt)

    return out.reshape(1, C, H, W)
